# Initial kernel scaffold; baseline (speedup 1.0000x reference)
import jax
import jax.numpy as jnp
from jax.experimental import pallas as pl


def kernel(idx_char, idx_bound, idx_flag, idx_radical, idx_pinyin, W_char, W_bound, W_flag, W_radical, W_pinyin):
    raise NotImplementedError("write your pallas kernel here")


# speedup vs baseline: 4.4913x; 4.4913x over previous
import jax
import jax.numpy as jnp
from jax import lax
from jax.experimental import pallas as pl
from jax.experimental.pallas import tpu as pltpu
from jax.experimental.pallas import tpu_sc as plsc

_TOT = 320
_B, _L = 4096, 50
_N = _B * _L
_C = 128
_NC, _NS = 2, 16
_NW = _NC * _NS
_PER_W = _N // _NW
_SUPER = 5
_INNER = 10
_IDX_CHUNK = _INNER * _C


def _sc_body(i0, i1, i2, i3, i4, w0, w1, w2, w3, w4, out,
             iv0, iv1, iv2, iv3, iv4, rc, rb, rf, rr, rp, asm1, rp64, sem):
    wid = lax.axis_index("s") * _NC + lax.axis_index("c")
    tok0 = wid * _PER_W
    idx_hbm = (i0, i1, i2, i3, i4)
    idx_v = (iv0, iv1, iv2, iv3, iv4)
    tables = (w0, w1, w2, w3, w4)
    dsts = (rc, rb, rf, rr, rp)

    def super_step(s, carry):
        stage0 = tok0 + s * _IDX_CHUNK
        for t in range(5):
            pltpu.sync_copy(idx_hbm[t].at[pl.ds(stage0, _IDX_CHUNK)], idx_v[t])

        def inner(j, carry2):
            off = j * _C
            waits = []
            for t in range(5):
                waits.append(pltpu.async_copy(
                    tables[t].at[idx_v[t].at[pl.ds(off, _C)]], dsts[t], sem))
            for wce in waits:
                wce.wait()

            def assemble(i, carry3):
                asm1[i, pl.ds(0, 16)] = rb[i, pl.ds(0, 16)]
                asm1[i, pl.ds(16, 16)] = rb[i, pl.ds(16, 16)]
                asm1[i, pl.ds(32, 16)] = rf[i, pl.ds(32, 16)]
                asm1[i, pl.ds(48, 16)] = rf[i, pl.ds(48, 16)]
                asm1[i, pl.ds(64, 16)] = rr[i, pl.ds(64, 16)]
                asm1[i, pl.ds(80, 16)] = rr[i, pl.ds(80, 16)]
                asm1[i, pl.ds(96, 16)] = rr[i, pl.ds(96, 16)]
                asm1[i, pl.ds(112, 16)] = rr[i, pl.ds(112, 16)]
                rp64[i, pl.ds(0, 16)] = rp[i, pl.ds(0, 16)]
                rp64[i, pl.ds(16, 16)] = rp[i, pl.ds(16, 16)]
                rp64[i, pl.ds(32, 16)] = rp[i, pl.ds(32, 16)]
                rp64[i, pl.ds(48, 16)] = rp[i, pl.ds(48, 16)]
                return carry3

            lax.fori_loop(0, _C, assemble, 0)
            row = stage0 + off
            pltpu.sync_copy(rc, out.at[pl.ds(row, _C), pl.ds(0, 128)])
            pltpu.sync_copy(asm1, out.at[pl.ds(row, _C), pl.ds(128, 128)])
            pltpu.sync_copy(rp64, out.at[pl.ds(row, _C), pl.ds(256, 64)])
            return carry2

        lax.fori_loop(0, _INNER, inner, 0)
        return carry

    lax.fori_loop(0, _SUPER, super_step, 0)


def _pad_cols(w, col0):
    v, d = w.shape
    return jnp.zeros((v, 128), jnp.float32).at[:, col0:col0 + d].set(w)


def kernel(idx_char, idx_bound, idx_flag, idx_radical, idx_pinyin,
           W_char, W_bound, W_flag, W_radical, W_pinyin):
    idxs = [a.reshape(_N).astype(jnp.int32)
            for a in (idx_char, idx_bound, idx_flag, idx_radical, idx_pinyin)]
    tables = [W_char,
              _pad_cols(W_bound, 0),
              _pad_cols(W_flag, 32),
              _pad_cols(W_radical, 64),
              _pad_cols(W_pinyin, 0)]
    scratch = ([pltpu.VMEM((_IDX_CHUNK,), jnp.int32) for _ in range(5)]
               + [pltpu.VMEM((_C, 128), jnp.float32) for _ in range(6)]
               + [pltpu.VMEM((_C, 64), jnp.float32)]
               + [pltpu.SemaphoreType.DMA])
    k = pl.kernel(
        _sc_body,
        out_type=jax.ShapeDtypeStruct((_N, _TOT), jnp.float32),
        mesh=plsc.VectorSubcoreMesh(core_axis_name="c", subcore_axis_name="s"),
        scratch_types=scratch,
    )
    out = k(*idxs, *tables)
    return out.reshape(_B, _L, _TOT)
